# baseline (speedup 1.0000x reference)
import functools

import jax
import jax.numpy as jnp
from jax import lax
from jax.experimental import pallas as pl
from jax.experimental.pallas import tpu as pltpu
from jax.experimental.pallas import tpu_sc as plsc

DIM = 32
N_CODES = 8192
N_TOKENS = 8192
TILE_M = 128
GRID = N_TOKENS // TILE_M
TABLE_W = 128
BLOCK_C = 2048


def _tc_body(x_ref, e_ref, inds_ref, loss_ref, et_ref, acc_ref):
    pid = pl.program_id(0)

    lane = lax.broadcasted_iota(jnp.int32, (TILE_M, DIM), 1)
    f = jnp.where(lane < DIM, x_ref[...], 0.0)
    e = e_ref[...]

    f2 = jnp.sum(f * f, axis=1, keepdims=True)
    e2 = jnp.sum(e * e, axis=0, keepdims=True)

    acc_v = acc_t = acc_i = None
    for b in range(N_CODES // BLOCK_C):
        e_b = e[:, b * BLOCK_C:(b + 1) * BLOCK_C]
        ab = jnp.dot(f, e_b, preferred_element_type=jnp.float32)
        dist = (f2 - 2.0 * ab) + e2[:, b * BLOCK_C:(b + 1) * BLOCK_C]
        bmin = jnp.min(dist, axis=1, keepdims=True)
        ids = lax.broadcasted_iota(jnp.int32, dist.shape, 1) + b * BLOCK_C
        bidx = jnp.min(jnp.where(dist == bmin, ids, N_CODES), axis=1)
        bmin = bmin[:, 0]
        if b == 0:
            acc_v = bmin.astype(jnp.bfloat16).astype(jnp.float32)
            acc_t = bmin
            acc_i = bidx
        else:
            keep = (acc_v < bmin) | ((acc_v == bmin) & (acc_i < bidx))
            acc_i = jnp.where(keep, acc_i, bidx)
            acc_t = jnp.where(keep, acc_t, bmin)
            acc_v = jnp.where(keep, acc_v, bmin)
            acc_v = acc_v.astype(jnp.bfloat16).astype(jnp.float32)
    inds_ref[...] = acc_i
    minval = acc_t

    @pl.when(pid == 0)
    def _init():
        acc_ref[0, 0] = 0.0
        et_ref[...] = jnp.concatenate(
            [e.T, jnp.zeros((N_CODES, TABLE_W - DIM), jnp.float32)], axis=1)

    acc_ref[0, 0] += jnp.sum(minval)

    @pl.when(pid == GRID - 1)
    def _fini():
        loss_ref[0, 0] = acc_ref[0, 0] * (1.25 / (N_TOKENS * DIM))


_tc_quantize = pl.pallas_call(
    _tc_body,
    grid=(GRID,),
    in_specs=[
        pl.BlockSpec((TILE_M, DIM), lambda i: (i, 0)),
        pl.BlockSpec((DIM, N_CODES), lambda i: (0, 0)),
    ],
    out_specs=[
        pl.BlockSpec((TILE_M,), lambda i: (i,)),
        pl.BlockSpec(memory_space=pltpu.SMEM),
        pl.BlockSpec((N_CODES, TABLE_W), lambda i: (0, 0)),
    ],
    out_shape=[
        jax.ShapeDtypeStruct((N_TOKENS,), jnp.int32),
        jax.ShapeDtypeStruct((1, 1), jnp.float32),
        jax.ShapeDtypeStruct((N_CODES, TABLE_W), jnp.float32),
    ],
    scratch_shapes=[pltpu.SMEM((1, 1), jnp.float32)],
    compiler_params=pltpu.CompilerParams(
        dimension_semantics=("arbitrary",),
    ),
)


_NC, _NS = 2, 16
_NW = _NC * _NS
_BPW = N_TOKENS // _NW
_CHUNK = 128
_NCHUNK = _BPW // _CHUNK

@functools.cache
def _make_sc_gather():
    mesh = plsc.VectorSubcoreMesh(core_axis_name="c", subcore_axis_name="s")

    @functools.partial(
        pl.kernel,
        mesh=mesh,
        out_type=jax.ShapeDtypeStruct((N_TOKENS, TABLE_W), jnp.float32),
        scratch_types=[
            pltpu.VMEM((_CHUNK,), jnp.int32),
            pltpu.VMEM((_CHUNK, TABLE_W), jnp.float32),
            pltpu.SemaphoreType.DMA,
        ],
    )
    def _sc_gather(table_hbm, idx_hbm, out_hbm, idx_v, rows_v, sem):
        wid = lax.axis_index("s") * _NC + lax.axis_index("c")
        base = wid * _BPW
        for j in range(_NCHUNK):
            off = base + j * _CHUNK
            pltpu.sync_copy(idx_hbm.at[pl.ds(off, _CHUNK)], idx_v)
            pltpu.async_copy(table_hbm.at[idx_v], rows_v, sem).wait()
            pltpu.sync_copy(rows_v, out_hbm.at[pl.ds(off, _CHUNK)])

    return _sc_gather


def kernel(input, embed):
    flatten = input.reshape(-1, DIM)
    inds, loss, table = _tc_quantize(flatten, embed)
    quantized = _make_sc_gather()(table, inds)[:, :DIM]
    return (
        quantized.reshape(input.shape),
        loss[0, 0],
        inds.reshape(input.shape[:-1]),
    )


# speedup vs baseline: 1.1420x; 1.1420x over previous
import functools

import jax
import jax.numpy as jnp
from jax import lax
from jax.experimental import pallas as pl
from jax.experimental.pallas import tpu as pltpu
from jax.experimental.pallas import tpu_sc as plsc

DIM = 32
N_CODES = 8192
N_TOKENS = 8192
TILE_M = 256
GRID = N_TOKENS // TILE_M
TABLE_W = 128
BLOCK_C = 2048


def _tc_body(x_ref, e_ref, inds_ref, loss_ref, et_ref, acc_ref, e2_ref):
    pid = pl.program_id(0)

    lane = lax.broadcasted_iota(jnp.int32, (TILE_M, DIM), 1)
    f = jnp.where(lane < DIM, x_ref[...], 0.0)
    e = e_ref[...]

    f2 = jnp.sum(f * f, axis=1, keepdims=True)
    g = -2.0 * f

    @pl.when(pid == 0)
    def _e2():
        e2_ref[...] = jnp.sum(e * e, axis=0, keepdims=True)

    e2 = e2_ref[...]

    acc_v = acc_t = acc_i = None
    for b in range(N_CODES // BLOCK_C):
        e_b = e[:, b * BLOCK_C:(b + 1) * BLOCK_C]
        ab2 = jnp.dot(g, e_b, preferred_element_type=jnp.float32)
        dist = (f2 + ab2) + e2[:, b * BLOCK_C:(b + 1) * BLOCK_C]
        bmin = jnp.min(dist, axis=1, keepdims=True)
        ids = lax.broadcasted_iota(jnp.int32, dist.shape, 1) + b * BLOCK_C
        bidx = jnp.min(jnp.where(dist == bmin, ids, N_CODES), axis=1)
        bmin = bmin[:, 0]
        if b == 0:
            acc_v = bmin.astype(jnp.bfloat16).astype(jnp.float32)
            acc_t = bmin
            acc_i = bidx
        else:
            keep = (acc_v < bmin) | ((acc_v == bmin) & (acc_i < bidx))
            acc_i = jnp.where(keep, acc_i, bidx)
            acc_t = jnp.where(keep, acc_t, bmin)
            acc_v = jnp.where(keep, acc_v, bmin)
            acc_v = acc_v.astype(jnp.bfloat16).astype(jnp.float32)
    inds_ref[...] = acc_i
    minval = acc_t

    @pl.when(pid == 0)
    def _init():
        acc_ref[0, 0] = 0.0
        et_ref[...] = jnp.concatenate(
            [e.T, jnp.zeros((N_CODES, TABLE_W - DIM), jnp.float32)], axis=1)

    acc_ref[0, 0] += jnp.sum(minval)

    @pl.when(pid == GRID - 1)
    def _fini():
        loss_ref[0, 0] = acc_ref[0, 0] * (1.25 / (N_TOKENS * DIM))


_tc_quantize = pl.pallas_call(
    _tc_body,
    grid=(GRID,),
    in_specs=[
        pl.BlockSpec((TILE_M, DIM), lambda i: (i, 0)),
        pl.BlockSpec((DIM, N_CODES), lambda i: (0, 0)),
    ],
    out_specs=[
        pl.BlockSpec((TILE_M,), lambda i: (i,)),
        pl.BlockSpec(memory_space=pltpu.SMEM),
        pl.BlockSpec((N_CODES, TABLE_W), lambda i: (0, 0)),
    ],
    out_shape=[
        jax.ShapeDtypeStruct((N_TOKENS,), jnp.int32),
        jax.ShapeDtypeStruct((1, 1), jnp.float32),
        jax.ShapeDtypeStruct((N_CODES, TABLE_W), jnp.float32),
    ],
    scratch_shapes=[pltpu.SMEM((1, 1), jnp.float32),
                    pltpu.VMEM((1, N_CODES), jnp.float32)],
    compiler_params=pltpu.CompilerParams(
        dimension_semantics=("arbitrary",),
    ),
)


_NC, _NS = 2, 16
_NW = _NC * _NS
_BPW = N_TOKENS // _NW
_CHUNK = 128
_NCHUNK = _BPW // _CHUNK

@functools.cache
def _make_sc_gather():
    mesh = plsc.VectorSubcoreMesh(core_axis_name="c", subcore_axis_name="s")

    @functools.partial(
        pl.kernel,
        mesh=mesh,
        out_type=jax.ShapeDtypeStruct((N_TOKENS, TABLE_W), jnp.float32),
        scratch_types=[
            pltpu.VMEM((_CHUNK,), jnp.int32),
            pltpu.VMEM((_CHUNK, TABLE_W), jnp.float32),
            pltpu.SemaphoreType.DMA,
        ],
    )
    def _sc_gather(table_hbm, idx_hbm, out_hbm, idx_v, rows_v, sem):
        wid = lax.axis_index("s") * _NC + lax.axis_index("c")
        base = wid * _BPW
        for j in range(_NCHUNK):
            off = base + j * _CHUNK
            pltpu.sync_copy(idx_hbm.at[pl.ds(off, _CHUNK)], idx_v)
            pltpu.async_copy(table_hbm.at[idx_v], rows_v, sem).wait()
            pltpu.sync_copy(rows_v, out_hbm.at[pl.ds(off, _CHUNK)])

    return _sc_gather


def kernel(input, embed):
    flatten = input.reshape(-1, DIM)
    inds, loss, table = _tc_quantize(flatten, embed)
    quantized = _make_sc_gather()(table, inds)[:, :DIM]
    return (
        quantized.reshape(input.shape),
        loss[0, 0],
        inds.reshape(input.shape[:-1]),
    )


# speedup vs baseline: 1.2091x; 1.0588x over previous
import functools

import jax
import jax.numpy as jnp
from jax import lax
from jax.experimental import pallas as pl
from jax.experimental.pallas import tpu as pltpu
from jax.experimental.pallas import tpu_sc as plsc

DIM = 32
N_CODES = 8192
N_TOKENS = 8192
TILE_M = 512
GRID = N_TOKENS // TILE_M
TABLE_W = 128
BLOCK_C = 2048


def _tc_body(x_ref, e_ref, inds_ref, loss_ref, et_ref, acc_ref, e2_ref):
    pid = pl.program_id(0)

    lane = lax.broadcasted_iota(jnp.int32, (TILE_M, DIM), 1)
    f = jnp.where(lane < DIM, x_ref[...], 0.0)
    e = e_ref[...]

    f2 = jnp.sum(f * f, axis=1, keepdims=True)
    g = -2.0 * f

    @pl.when(pid == 0)
    def _e2():
        e2_ref[...] = jnp.sum(e * e, axis=0, keepdims=True)

    e2 = e2_ref[...]

    acc_v = acc_t = acc_i = None
    for b in range(N_CODES // BLOCK_C):
        e_b = e[:, b * BLOCK_C:(b + 1) * BLOCK_C]
        ab2 = jnp.dot(g, e_b, preferred_element_type=jnp.float32)
        dist = (f2 + ab2) + e2[:, b * BLOCK_C:(b + 1) * BLOCK_C]
        bmin = jnp.min(dist, axis=1, keepdims=True)
        ids = lax.broadcasted_iota(jnp.int32, dist.shape, 1) + b * BLOCK_C
        bidx = jnp.min(jnp.where(dist == bmin, ids, N_CODES), axis=1)
        bmin = bmin[:, 0]
        if b == 0:
            acc_v = bmin.astype(jnp.bfloat16).astype(jnp.float32)
            acc_t = bmin
            acc_i = bidx
        else:
            keep = (acc_v < bmin) | ((acc_v == bmin) & (acc_i < bidx))
            acc_i = jnp.where(keep, acc_i, bidx)
            acc_t = jnp.where(keep, acc_t, bmin)
            acc_v = jnp.where(keep, acc_v, bmin)
            acc_v = acc_v.astype(jnp.bfloat16).astype(jnp.float32)
    inds_ref[...] = acc_i
    minval = acc_t

    @pl.when(pid == 0)
    def _init():
        acc_ref[0, 0] = 0.0
        et_ref[...] = jnp.concatenate(
            [e.T, jnp.zeros((N_CODES, TABLE_W - DIM), jnp.float32)], axis=1)

    acc_ref[0, 0] += jnp.sum(minval)

    @pl.when(pid == GRID - 1)
    def _fini():
        loss_ref[0, 0] = acc_ref[0, 0] * (1.25 / (N_TOKENS * DIM))


_tc_quantize = pl.pallas_call(
    _tc_body,
    grid=(GRID,),
    in_specs=[
        pl.BlockSpec((TILE_M, DIM), lambda i: (i, 0)),
        pl.BlockSpec((DIM, N_CODES), lambda i: (0, 0)),
    ],
    out_specs=[
        pl.BlockSpec((TILE_M,), lambda i: (i,)),
        pl.BlockSpec(memory_space=pltpu.SMEM),
        pl.BlockSpec((N_CODES, TABLE_W), lambda i: (0, 0)),
    ],
    out_shape=[
        jax.ShapeDtypeStruct((N_TOKENS,), jnp.int32),
        jax.ShapeDtypeStruct((1, 1), jnp.float32),
        jax.ShapeDtypeStruct((N_CODES, TABLE_W), jnp.float32),
    ],
    scratch_shapes=[pltpu.SMEM((1, 1), jnp.float32),
                    pltpu.VMEM((1, N_CODES), jnp.float32)],
    compiler_params=pltpu.CompilerParams(
        dimension_semantics=("arbitrary",),
    ),
)


_NC, _NS = 2, 16
_NW = _NC * _NS
_BPW = N_TOKENS // _NW
_CHUNK = 128
_NCHUNK = _BPW // _CHUNK

@functools.cache
def _make_sc_gather():
    mesh = plsc.VectorSubcoreMesh(core_axis_name="c", subcore_axis_name="s")

    @functools.partial(
        pl.kernel,
        mesh=mesh,
        out_type=jax.ShapeDtypeStruct((N_TOKENS, TABLE_W), jnp.float32),
        scratch_types=[
            pltpu.VMEM((_CHUNK,), jnp.int32),
            pltpu.VMEM((_CHUNK, TABLE_W), jnp.float32),
            pltpu.SemaphoreType.DMA,
        ],
    )
    def _sc_gather(table_hbm, idx_hbm, out_hbm, idx_v, rows_v, sem):
        wid = lax.axis_index("s") * _NC + lax.axis_index("c")
        base = wid * _BPW
        for j in range(_NCHUNK):
            off = base + j * _CHUNK
            pltpu.sync_copy(idx_hbm.at[pl.ds(off, _CHUNK)], idx_v)
            pltpu.async_copy(table_hbm.at[idx_v], rows_v, sem).wait()
            pltpu.sync_copy(rows_v, out_hbm.at[pl.ds(off, _CHUNK)])

    return _sc_gather


def kernel(input, embed):
    flatten = input.reshape(-1, DIM)
    inds, loss, table = _tc_quantize(flatten, embed)
    quantized = _make_sc_gather()(table, inds)[:, :DIM]
    return (
        quantized.reshape(input.shape),
        loss[0, 0],
        inds.reshape(input.shape[:-1]),
    )


# speedup vs baseline: 1.2659x; 1.0470x over previous
import functools

import jax
import jax.numpy as jnp
from jax import lax
from jax.experimental import pallas as pl
from jax.experimental.pallas import tpu as pltpu
from jax.experimental.pallas import tpu_sc as plsc

DIM = 32
N_CODES = 8192
N_TOKENS = 8192
TILE_M = 1024
GRID = N_TOKENS // TILE_M
TABLE_W = 128
BLOCK_C = 2048


def _tc_body(x_ref, e_ref, inds_ref, loss_ref, et_ref, acc_ref, e2_ref):
    pid = pl.program_id(0)

    lane = lax.broadcasted_iota(jnp.int32, (TILE_M, DIM), 1)
    f = jnp.where(lane < DIM, x_ref[...], 0.0)
    e = e_ref[...]

    f2 = jnp.sum(f * f, axis=1, keepdims=True)
    g = -2.0 * f

    @pl.when(pid == 0)
    def _e2():
        e2_ref[...] = jnp.sum(e * e, axis=0, keepdims=True)

    e2 = e2_ref[...]

    acc_v = acc_t = acc_i = None
    for b in range(N_CODES // BLOCK_C):
        e_b = e[:, b * BLOCK_C:(b + 1) * BLOCK_C]
        ab2 = jnp.dot(g, e_b, preferred_element_type=jnp.float32)
        dist = (f2 + ab2) + e2[:, b * BLOCK_C:(b + 1) * BLOCK_C]
        bmin = jnp.min(dist, axis=1, keepdims=True)
        ids = lax.broadcasted_iota(jnp.int32, dist.shape, 1) + b * BLOCK_C
        bidx = jnp.min(jnp.where(dist == bmin, ids, N_CODES), axis=1)
        bmin = bmin[:, 0]
        if b == 0:
            acc_v = bmin.astype(jnp.bfloat16).astype(jnp.float32)
            acc_t = bmin
            acc_i = bidx
        else:
            keep = (acc_v < bmin) | ((acc_v == bmin) & (acc_i < bidx))
            acc_i = jnp.where(keep, acc_i, bidx)
            acc_t = jnp.where(keep, acc_t, bmin)
            acc_v = jnp.where(keep, acc_v, bmin)
            acc_v = acc_v.astype(jnp.bfloat16).astype(jnp.float32)
    inds_ref[...] = acc_i
    minval = acc_t

    @pl.when(pid == 0)
    def _init():
        acc_ref[0, 0] = 0.0
        et_ref[...] = jnp.concatenate(
            [e.T, jnp.zeros((N_CODES, TABLE_W - DIM), jnp.float32)], axis=1)

    acc_ref[0, 0] += jnp.sum(minval)

    @pl.when(pid == GRID - 1)
    def _fini():
        loss_ref[0, 0] = acc_ref[0, 0] * (1.25 / (N_TOKENS * DIM))


_tc_quantize = pl.pallas_call(
    _tc_body,
    grid=(GRID,),
    in_specs=[
        pl.BlockSpec((TILE_M, DIM), lambda i: (i, 0)),
        pl.BlockSpec((DIM, N_CODES), lambda i: (0, 0)),
    ],
    out_specs=[
        pl.BlockSpec((TILE_M,), lambda i: (i,)),
        pl.BlockSpec(memory_space=pltpu.SMEM),
        pl.BlockSpec((N_CODES, TABLE_W), lambda i: (0, 0)),
    ],
    out_shape=[
        jax.ShapeDtypeStruct((N_TOKENS,), jnp.int32),
        jax.ShapeDtypeStruct((1, 1), jnp.float32),
        jax.ShapeDtypeStruct((N_CODES, TABLE_W), jnp.float32),
    ],
    scratch_shapes=[pltpu.SMEM((1, 1), jnp.float32),
                    pltpu.VMEM((1, N_CODES), jnp.float32)],
    compiler_params=pltpu.CompilerParams(
        dimension_semantics=("arbitrary",),
    ),
)


_NC, _NS = 2, 16
_NW = _NC * _NS
_BPW = N_TOKENS // _NW
_CHUNK = 128
_NCHUNK = _BPW // _CHUNK

@functools.cache
def _make_sc_gather():
    mesh = plsc.VectorSubcoreMesh(core_axis_name="c", subcore_axis_name="s")

    @functools.partial(
        pl.kernel,
        mesh=mesh,
        out_type=jax.ShapeDtypeStruct((N_TOKENS, TABLE_W), jnp.float32),
        scratch_types=[
            pltpu.VMEM((_CHUNK,), jnp.int32),
            pltpu.VMEM((_CHUNK, TABLE_W), jnp.float32),
            pltpu.SemaphoreType.DMA,
        ],
    )
    def _sc_gather(table_hbm, idx_hbm, out_hbm, idx_v, rows_v, sem):
        wid = lax.axis_index("s") * _NC + lax.axis_index("c")
        base = wid * _BPW
        for j in range(_NCHUNK):
            off = base + j * _CHUNK
            pltpu.sync_copy(idx_hbm.at[pl.ds(off, _CHUNK)], idx_v)
            pltpu.async_copy(table_hbm.at[idx_v], rows_v, sem).wait()
            pltpu.sync_copy(rows_v, out_hbm.at[pl.ds(off, _CHUNK)])

    return _sc_gather


def kernel(input, embed):
    flatten = input.reshape(-1, DIM)
    inds, loss, table = _tc_quantize(flatten, embed)
    quantized = _make_sc_gather()(table, inds)[:, :DIM]
    return (
        quantized.reshape(input.shape),
        loss[0, 0],
        inds.reshape(input.shape[:-1]),
    )
